# Initial kernel scaffold; baseline (speedup 1.0000x reference)
import jax
import jax.numpy as jnp
from jax.experimental import pallas as pl


def kernel(x, pi_logits, mus, log_sigmas, Ws1, bs1, Ws2, bs2, Wc1, bc1, Wc2, bc2):
    raise NotImplementedError("write your pallas kernel here")


# speedup vs baseline: 2.7337x; 2.7337x over previous
import functools
import math

import jax
import jax.numpy as jnp
from jax.experimental import pallas as pl
from jax.experimental.pallas import tpu as pltpu

T = 8
D = 1024
H = 512
NS = 2
N = 2048
TILE = 512
DH = D // 2
LOG2PI = math.log(2.0 * math.pi)


def _flow_kernel(x_ref, pi_ref, mus_ref, ls_ref,
                 Ws1_ref, bs1_ref, Ws2_ref, bs2_ref,
                 Wc1_ref, bc1_ref, Wc2_ref, bc2_ref,
                 out_ref,
                 A_ref, B_ref, lds_ref, lk_ref):
    k = pl.program_id(1)
    x1 = x_ref[:, :DH]

    @pl.when(k == 0)
    def _shared():
        h1 = jnp.tanh(jnp.dot(x1, Ws1_ref[1], preferred_element_type=jnp.float32)
                      + bs1_ref[1][None, :])
        st1 = jnp.dot(h1, Ws2_ref[1], preferred_element_type=jnp.float32) + bs2_ref[1][None, :]
        s1 = jnp.tanh(st1[:, :DH])
        t1 = st1[:, DH:]
        h0 = jnp.tanh(jnp.dot(x1, Ws1_ref[0], preferred_element_type=jnp.float32)
                      + bs1_ref[0][None, :])
        st0 = jnp.dot(h0, Ws2_ref[0], preferred_element_type=jnp.float32) + bs2_ref[0][None, :]
        s0 = jnp.tanh(st0[:, :DH])
        t0 = st0[:, DH:]
        e0 = jnp.exp(-s0)
        A = jnp.exp(-(s0 + s1))
        A_ref[:, :] = A
        B_ref[:, :] = -(t1 * A + t0 * e0)
        lds_ref[0, :] = -jnp.sum(s0 + s1, axis=1)

    h = jnp.tanh(jnp.dot(x1, Wc1_ref[0], preferred_element_type=jnp.float32)
                 + bc1_ref[0])
    st = jnp.dot(h, Wc2_ref[0], preferred_element_type=jnp.float32) + bc2_ref[0]
    s = jnp.tanh(st[:, :DH])
    t = st[:, DH:]
    x2 = x_ref[:, DH:]
    z2 = (x2 - t) * jnp.exp(-s) * A_ref[:, :] + B_ref[:, :]

    mu = mus_ref[0, 0]
    ls = ls_ref[0, 0]
    r1 = (x1 - mu[None, :DH]) * jnp.exp(-ls[None, :DH])
    r2 = (z2 - mu[None, DH:]) * jnp.exp(-ls[None, DH:])
    g = -0.5 * (jnp.sum(r1 * r1, axis=1) + jnp.sum(r2 * r2, axis=1)
                + 2.0 * jnp.sum(ls) + D * LOG2PI)
    lk_ref[pl.ds(k, 1), :] = (g - jnp.sum(s, axis=1))[None, :]

    @pl.when(k == T - 1)
    def _finish():
        pi = pi_ref[0, :]
        la = pi - (jnp.max(pi) + jnp.log(jnp.sum(jnp.exp(pi - jnp.max(pi)))))
        lp = lk_ref[:, :] + la[:, None]
        m = jnp.max(lp, axis=0)
        lse = m + jnp.log(jnp.sum(jnp.exp(lp - m[None, :]), axis=0))
        out_ref[0, :] = lse + lds_ref[0, :]


@jax.jit
def kernel(x, pi_logits, mus, log_sigmas, Ws1, bs1, Ws2, bs2, Wc1, bc1, Wc2, bc2):
    n = x.shape[0]
    Wc1r = Wc1.reshape(T, DH, H)
    bc1r = bc1.reshape(T, 1, H)
    Wc2r = Wc2.reshape(T, H, D)
    bc2r = bc2.reshape(T, 1, D)
    mus3 = mus.reshape(T, 1, D)
    ls3 = log_sigmas.reshape(T, 1, D)
    pi2 = pi_logits.reshape(1, T)

    grid = (n // TILE, T)
    out = pl.pallas_call(
        _flow_kernel,
        grid=grid,
        in_specs=[
            pl.BlockSpec((TILE, D), lambda i, k: (i, 0)),
            pl.BlockSpec((1, T), lambda i, k: (0, 0)),
            pl.BlockSpec((1, 1, D), lambda i, k: (k, 0, 0)),
            pl.BlockSpec((1, 1, D), lambda i, k: (k, 0, 0)),
            pl.BlockSpec((NS, DH, H), lambda i, k: (0, 0, 0)),
            pl.BlockSpec((NS, H), lambda i, k: (0, 0)),
            pl.BlockSpec((NS, H, D), lambda i, k: (0, 0, 0)),
            pl.BlockSpec((NS, D), lambda i, k: (0, 0)),
            pl.BlockSpec((1, DH, H), lambda i, k: (k, 0, 0)),
            pl.BlockSpec((1, 1, H), lambda i, k: (k, 0, 0)),
            pl.BlockSpec((1, H, D), lambda i, k: (k, 0, 0)),
            pl.BlockSpec((1, 1, D), lambda i, k: (k, 0, 0)),
        ],
        out_specs=pl.BlockSpec((1, TILE), lambda i, k: (0, i)),
        out_shape=jax.ShapeDtypeStruct((1, n), jnp.float32),
        scratch_shapes=[
            pltpu.VMEM((TILE, DH), jnp.float32),
            pltpu.VMEM((TILE, DH), jnp.float32),
            pltpu.VMEM((1, TILE), jnp.float32),
            pltpu.VMEM((T, TILE), jnp.float32),
        ],
    )(x, pi2, mus3, ls3, Ws1, bs1, Ws2, bs2, Wc1r, bc1r, Wc2r, bc2r)
    return out[0]
